# Initial kernel scaffold; baseline (speedup 1.0000x reference)
import jax
import jax.numpy as jnp
from jax.experimental import pallas as pl
from jax.experimental.pallas import tpu as pltpu

R = 2048
C = 128
N = R * C
P = 128


def _bitonic_sort(a, ir, ic):
    for k in range(1, 19):
        if k <= 10:
            asc = ((ir >> k) & 1) == 0
        elif k <= 17:
            asc = ((ic >> (k - 11)) & 1) == 0
        else:
            asc = None
        for j in range(k - 1, -1, -1):
            if j <= 10:
                s = 1 << j
                up = jnp.roll(a, -s, axis=0)
                dn = jnp.roll(a, s, axis=0)
                low = ((ir >> j) & 1) == 0
            else:
                L = 1 << (j - 11)
                up = jnp.roll(a, -L, axis=1)
                dn = jnp.roll(a, L, axis=1)
                low = ((ic >> (j - 11)) & 1) == 0
            partner = jnp.where(low, up, dn)
            mn = jnp.minimum(a, partner)
            mx = jnp.maximum(a, partner)
            keep_min = low if asc is None else (low == asc)
            a = jnp.where(keep_min, mn, mx)
    return a


def _body(dirs_ref, w_ref, t_ref, out_ref):
    p = pl.program_id(0)
    d0 = dirs_ref[p, 0]
    d1 = dirs_ref[p, 1]
    d2 = dirs_ref[p, 2]
    nrm = jnp.maximum(jnp.sqrt(d0 * d0 + d1 * d1 + d2 * d2), 1e-12)
    e0 = d0 / nrm
    e1 = d1 / nrm
    e2 = d2 / nrm

    px = e0 * w_ref[0] + e1 * w_ref[1] + e2 * w_ref[2]
    py = e0 * t_ref[0] + e1 * t_ref[1] + e2 * t_ref[2]

    ir = jax.lax.broadcasted_iota(jnp.int32, (R, C), 0)
    ic = jax.lax.broadcasted_iota(jnp.int32, (R, C), 1)
    sx = _bitonic_sort(px, ir, ic)
    sy = _bitonic_sort(py, ir, ic)

    part = jnp.sum((sx - sy) ** 2)
    prev = jnp.where(p == 0, 0.0, out_ref[0, 0])
    total = prev + part
    out_ref[0, 0] = jnp.where(p == P - 1, total / (float(N) * float(P)), total)


def kernel(warped, target, directions):
    pix_w = warped.reshape(3, R, C)
    pix_t = target.reshape(3, R, C)
    out = pl.pallas_call(
        _body,
        grid=(P,),
        in_specs=[
            pl.BlockSpec(memory_space=pltpu.SMEM),
            pl.BlockSpec((3, R, C), lambda p: (0, 0, 0)),
            pl.BlockSpec((3, R, C), lambda p: (0, 0, 0)),
        ],
        out_specs=pl.BlockSpec(memory_space=pltpu.SMEM),
        out_shape=jax.ShapeDtypeStruct((1, 1), jnp.float32),
        compiler_params=pltpu.CompilerParams(
            dimension_semantics=("arbitrary",),
        ),
    )(directions, pix_w, pix_t)
    return out[0, 0]


# speedup vs baseline: 3.2882x; 3.2882x over previous
import jax
import jax.numpy as jnp
from jax.experimental import pallas as pl
from jax.experimental.pallas import tpu as pltpu

R = 2048
C = 128
N = R * C
P = 128


def _bitonic_sort(ref, ir, ic):
    for k in range(1, 19):
        if k <= 10:
            asc = ((ir >> k) & 1) == 0
        elif k <= 17:
            asc = ((ic >> (k - 11)) & 1) == 0
        else:
            asc = None
        for j in range(k - 1, -1, -1):
            a = ref[...]
            if j <= 10:
                s = 1 << j
                up = jnp.roll(a, -s, axis=0)
                dn = jnp.roll(a, s, axis=0)
                low = ((ir >> j) & 1) == 0
            else:
                L = 1 << (j - 11)
                up = jnp.roll(a, -L, axis=1)
                dn = jnp.roll(a, L, axis=1)
                low = ((ic >> (j - 11)) & 1) == 0
            partner = jnp.where(low, up, dn)
            mn = jnp.minimum(a, partner)
            mx = jnp.maximum(a, partner)
            if asc is None:
                ref[...] = jnp.where(low, mn, mx)
            else:
                ref[...] = jnp.where(low, jnp.where(asc, mn, mx),
                                     jnp.where(asc, mx, mn))


def _body(dirs_ref, w_ref, t_ref, out_ref, sx_ref, sy_ref):
    p = pl.program_id(0)
    d0 = dirs_ref[p, 0]
    d1 = dirs_ref[p, 1]
    d2 = dirs_ref[p, 2]
    nrm = jnp.maximum(jnp.sqrt(d0 * d0 + d1 * d1 + d2 * d2), 1e-12)
    e0 = d0 / nrm
    e1 = d1 / nrm
    e2 = d2 / nrm

    sx_ref[...] = e0 * w_ref[0] + e1 * w_ref[1] + e2 * w_ref[2]
    sy_ref[...] = e0 * t_ref[0] + e1 * t_ref[1] + e2 * t_ref[2]

    ir = jax.lax.broadcasted_iota(jnp.int32, (R, 1), 0)
    ic = jax.lax.broadcasted_iota(jnp.int32, (1, C), 1)
    _bitonic_sort(sx_ref, ir, ic)
    _bitonic_sort(sy_ref, ir, ic)

    part = jnp.sum((sx_ref[...] - sy_ref[...]) ** 2)
    prev = jnp.where(p == 0, 0.0, out_ref[0, 0])
    total = prev + part
    out_ref[0, 0] = jnp.where(p == P - 1, total / (float(N) * float(P)), total)


def kernel(warped, target, directions):
    pix_w = warped.reshape(3, R, C)
    pix_t = target.reshape(3, R, C)
    out = pl.pallas_call(
        _body,
        grid=(P,),
        in_specs=[
            pl.BlockSpec(memory_space=pltpu.SMEM),
            pl.BlockSpec((3, R, C), lambda p: (0, 0, 0)),
            pl.BlockSpec((3, R, C), lambda p: (0, 0, 0)),
        ],
        out_specs=pl.BlockSpec(memory_space=pltpu.SMEM),
        out_shape=jax.ShapeDtypeStruct((1, 1), jnp.float32),
        scratch_shapes=[
            pltpu.VMEM((R, C), jnp.float32),
            pltpu.VMEM((R, C), jnp.float32),
        ],
        compiler_params=pltpu.CompilerParams(
            dimension_semantics=("arbitrary",),
        ),
    )(directions, pix_w, pix_t)
    return out[0, 0]


# speedup vs baseline: 4.9456x; 1.5041x over previous
import jax
import jax.numpy as jnp
from jax.experimental import pallas as pl
from jax.experimental.pallas import tpu as pltpu

R = 2048
C = 128
N = R * C
P = 128


def _bitonic_sort(ref, ir, ic):
    rows = ref.shape[0]
    for k in range(1, 19):
        if k <= 10:
            asc = ((ir >> k) & 1) == 0
        elif k <= 17:
            asc = ((ic >> (k - 11)) & 1) == 0
        else:
            asc = None
        for j in range(k - 1, -1, -1):
            a = ref[...]
            if 3 <= j <= 10:
                s = 1 << j
                m = rows // (2 * s)
                v = a.reshape(m, 2, s, C)
                lo = v[:, 0]
                hi = v[:, 1]
                mn = jnp.minimum(lo, hi)
                mx = jnp.maximum(lo, hi)
                if asc is None:
                    nl, nh = mn, mx
                else:
                    if k <= 10:
                        io = jax.lax.broadcasted_iota(jnp.int32, (m, 1, 1), 0)
                        ascb = ((io >> (k - j - 1)) & 1) == 0
                    else:
                        ascb = asc.reshape(1, 1, C)
                    nl = jnp.where(ascb, mn, mx)
                    nh = jnp.where(ascb, mx, mn)
                ref[...] = jnp.stack([nl, nh], axis=1).reshape(rows, C)
                continue
            if j <= 2:
                s = 1 << j
                up = jnp.roll(a, -s, axis=0)
                dn = jnp.roll(a, s, axis=0)
                low = ((ir >> j) & 1) == 0
            else:
                L = 1 << (j - 11)
                up = jnp.roll(a, -L, axis=1)
                dn = jnp.roll(a, L, axis=1)
                low = ((ic >> (j - 11)) & 1) == 0
            partner = jnp.where(low, up, dn)
            mn = jnp.minimum(a, partner)
            mx = jnp.maximum(a, partner)
            if asc is None:
                ref[...] = jnp.where(low, mn, mx)
            else:
                ref[...] = jnp.where(low, jnp.where(asc, mn, mx),
                                     jnp.where(asc, mx, mn))


def _body(dirs_ref, w_ref, t_ref, out_ref, s_ref):
    p = pl.program_id(0)
    d0 = dirs_ref[p, 0]
    d1 = dirs_ref[p, 1]
    d2 = dirs_ref[p, 2]
    nrm = jnp.maximum(jnp.sqrt(d0 * d0 + d1 * d1 + d2 * d2), 1e-12)
    e0 = d0 / nrm
    e1 = d1 / nrm
    e2 = d2 / nrm

    s_ref[0:R] = e0 * w_ref[0] + e1 * w_ref[1] + e2 * w_ref[2]
    s_ref[R:2 * R] = e0 * t_ref[0] + e1 * t_ref[1] + e2 * t_ref[2]

    ir = jax.lax.broadcasted_iota(jnp.int32, (2 * R, 1), 0)
    ic = jax.lax.broadcasted_iota(jnp.int32, (1, C), 1)
    _bitonic_sort(s_ref, ir, ic)

    part = jnp.sum((s_ref[0:R] - s_ref[R:2 * R]) ** 2)
    prev = jnp.where(p == 0, 0.0, out_ref[0, 0])
    total = prev + part
    out_ref[0, 0] = jnp.where(p == P - 1, total / (float(N) * float(P)), total)


def kernel(warped, target, directions):
    pix_w = warped.reshape(3, R, C)
    pix_t = target.reshape(3, R, C)
    out = pl.pallas_call(
        _body,
        grid=(P,),
        in_specs=[
            pl.BlockSpec(memory_space=pltpu.SMEM),
            pl.BlockSpec((3, R, C), lambda p: (0, 0, 0)),
            pl.BlockSpec((3, R, C), lambda p: (0, 0, 0)),
        ],
        out_specs=pl.BlockSpec(memory_space=pltpu.SMEM),
        out_shape=jax.ShapeDtypeStruct((1, 1), jnp.float32),
        scratch_shapes=[
            pltpu.VMEM((2 * R, C), jnp.float32),
        ],
        compiler_params=pltpu.CompilerParams(
            dimension_semantics=("arbitrary",),
        ),
    )(directions, pix_w, pix_t)
    return out[0, 0]
